# baseline (speedup 1.0000x reference)
import functools

import jax
import jax.numpy as jnp
from jax import lax
from jax.experimental import pallas as pl
from jax.experimental.pallas import tpu as pltpu
from jax.experimental.pallas import tpu_sc as plsc

_E = 8
_KSEL = 2
_TOK = 2048
_D = 2048
_F = 4096
_P = _TOK * _KSEL

_TM = 256
_TPAD = 24
_PP = _TPAD * _TM
_FB = 2048
_JA = _F // _FB
_DB = 1024
_JB = _D // _DB


def _route(topK_indices):
    flat = topK_indices.reshape(-1).astype(jnp.int32)
    perm = jnp.argsort(flat, stable=True).astype(jnp.int32)
    srcrow = perm // _KSEL
    pos = jnp.zeros((_P,), jnp.int32).at[perm].set(
        jnp.arange(_P, dtype=jnp.int32))
    counts = jnp.bincount(flat, length=_E).astype(jnp.int32)
    cum = jnp.cumsum(counts)
    start = jnp.concatenate([jnp.zeros((1,), jnp.int32),
                             cum[:-1].astype(jnp.int32)])
    ptiles = (counts + _TM - 1) // _TM
    pcum = jnp.cumsum(ptiles)
    pstart = (jnp.concatenate([jnp.zeros((1,), jnp.int32),
                               pcum[:-1].astype(jnp.int32)]) * _TM)
    tids = jnp.arange(_TPAD, dtype=jnp.int32)
    owner = jnp.minimum(jnp.searchsorted(pcum, tids, side="right"),
                        _E - 1).astype(jnp.int32)
    p_ids = jnp.arange(_P, dtype=jnp.int32)
    e_of_p = jnp.searchsorted(cum, p_ids, side="right").astype(jnp.int32)
    e_of_p = jnp.minimum(e_of_p, _E - 1)
    pp = pstart[e_of_p] + (p_ids - start[e_of_p])
    src_pad = jnp.zeros((_PP,), jnp.int32).at[pp].set(srcrow)
    pos_pad = pp[pos]
    return perm, src_pad, pos_pad, owner, pp


def _sc_gather(x, src_pad):
    info = plsc.get_sparse_core_info()
    nw = info.num_cores * info.num_subcores
    bpw = _PP // nw
    ch = 16
    nchunk = bpw // ch
    mesh = plsc.VectorSubcoreMesh(core_axis_name="c", subcore_axis_name="s")

    @functools.partial(
        pl.kernel, mesh=mesh,
        out_type=jax.ShapeDtypeStruct((_PP, _D), jnp.float32),
        scratch_types=[
            pltpu.VMEM((bpw,), jnp.int32),
            pltpu.VMEM((ch, _D), jnp.float32),
            pltpu.VMEM((ch, _D), jnp.float32),
            pltpu.SemaphoreType.DMA,
            pltpu.SemaphoreType.DMA,
        ],
    )
    def k(x_hbm, idx_hbm, out_hbm, idx_v, buf0, buf1, sem0, sem1):
        wid = lax.axis_index("s") * info.num_cores + lax.axis_index("c")
        base = wid * bpw
        pltpu.sync_copy(idx_hbm.at[pl.ds(base, bpw)], idx_v)
        bufs = (buf0, buf1)
        sems = (sem0, sem1)
        pltpu.async_copy(x_hbm.at[idx_v.at[pl.ds(0, ch)]], buf0, sem0)

        def body(i, _):
            sl = i % 2

            @pl.when(i + 1 < nchunk)
            def _():
                nsl = (i + 1) % 2
                for b in range(2):
                    @pl.when(nsl == b)
                    def _():
                        pltpu.async_copy(
                            x_hbm.at[idx_v.at[pl.ds((i + 1) * ch, ch)]],
                            bufs[b], sems[b])

            for b in range(2):
                @pl.when(sl == b)
                def _():
                    pltpu.make_async_copy(
                        x_hbm.at[idx_v.at[pl.ds(0, ch)]], bufs[b], sems[b]).wait()
                    pltpu.sync_copy(bufs[b], out_hbm.at[pl.ds(base + i * ch, ch)])
            return 0

        lax.fori_loop(0, nchunk, body, 0)

    return k(x, src_pad)


def _a_body(ow_ref, x_ref, w1_ref, b1_ref, h_ref):
    x = x_ref[...].astype(jnp.bfloat16)
    h = jnp.dot(x, w1_ref[0].astype(jnp.bfloat16),
                preferred_element_type=jnp.float32)
    h = jnp.maximum(h + b1_ref[0], 0.0)
    h_ref[...] = h.astype(jnp.bfloat16)


def _tc_phase_a(xs, W1, b1):
    grid_spec = pltpu.PrefetchScalarGridSpec(
        num_scalar_prefetch=1,
        grid=(_JA, _TPAD),
        in_specs=[
            pl.BlockSpec((_TM, _D), lambda j, t, ow: (t, 0)),
            pl.BlockSpec((1, _D, _FB), lambda j, t, ow: (ow[t], 0, j)),
            pl.BlockSpec((1, 1, _FB), lambda j, t, ow: (ow[t], 0, j)),
        ],
        out_specs=pl.BlockSpec((_TM, _FB), lambda j, t, ow: (t, j)),
    )
    return grid_spec


def _run_phase_a(xs, W1, b1, owner):
    return pl.pallas_call(
        _a_body,
        grid_spec=_tc_phase_a(xs, W1, b1),
        out_shape=jax.ShapeDtypeStruct((_PP, _F), jnp.bfloat16),
        compiler_params=pltpu.CompilerParams(
            dimension_semantics=("arbitrary", "arbitrary")),
    )(owner, xs, W1, b1.reshape(_E, 1, _F))


def _b_body(ow_ref, h_ref, w2_ref, b2_ref, sc_ref, o_ref):
    h = h_ref[...]
    o = jnp.dot(h, w2_ref[0].astype(jnp.bfloat16),
                preferred_element_type=jnp.float32)
    o_ref[...] = (o + b2_ref[0]) * sc_ref[...]


def _run_phase_b(H, W2, b2, scores_pad, owner):
    grid_spec = pltpu.PrefetchScalarGridSpec(
        num_scalar_prefetch=1,
        grid=(_JB, _TPAD),
        in_specs=[
            pl.BlockSpec((_TM, _F), lambda d, t, ow: (t, 0)),
            pl.BlockSpec((1, _F, _DB), lambda d, t, ow: (ow[t], 0, d)),
            pl.BlockSpec((1, 1, _DB), lambda d, t, ow: (ow[t], 0, d)),
            pl.BlockSpec((_TM, 1), lambda d, t, ow: (t, 0)),
        ],
        out_specs=pl.BlockSpec((_TM, _DB), lambda d, t, ow: (t, d)),
    )
    return pl.pallas_call(
        _b_body,
        grid_spec=grid_spec,
        out_shape=jax.ShapeDtypeStruct((_PP, _D), jnp.float32),
        compiler_params=pltpu.CompilerParams(
            dimension_semantics=("arbitrary", "arbitrary")),
    )(owner, H, W2, b2.reshape(_E, 1, _D), scores_pad)


def _sc_combine(outs, pos_even, pos_odd):
    info = plsc.get_sparse_core_info()
    nw = info.num_cores * info.num_subcores
    tpw = _TOK // nw
    ch = 16
    nchunk = tpw // ch
    lanes = _D // 16
    mesh = plsc.VectorSubcoreMesh(core_axis_name="c", subcore_axis_name="s")

    @functools.partial(
        pl.kernel, mesh=mesh,
        out_type=jax.ShapeDtypeStruct((_TOK, _D), jnp.float32),
        scratch_types=[
            pltpu.VMEM((tpw,), jnp.int32),
            pltpu.VMEM((tpw,), jnp.int32),
            pltpu.VMEM((ch, _D), jnp.float32),
            pltpu.VMEM((ch, _D), jnp.float32),
            pltpu.SemaphoreType.DMA,
            pltpu.SemaphoreType.DMA,
        ],
    )
    def k(rows_hbm, pe_hbm, po_hbm, y_hbm, pe_v, po_v, bufa, bufb, sema, semb):
        wid = lax.axis_index("s") * info.num_cores + lax.axis_index("c")
        base = wid * tpw
        pltpu.sync_copy(pe_hbm.at[pl.ds(base, tpw)], pe_v)
        pltpu.sync_copy(po_hbm.at[pl.ds(base, tpw)], po_v)

        def body(i, _):
            cpa = pltpu.async_copy(
                rows_hbm.at[pe_v.at[pl.ds(i * ch, ch)]], bufa, sema)
            cpb = pltpu.async_copy(
                rows_hbm.at[po_v.at[pl.ds(i * ch, ch)]], bufb, semb)
            cpa.wait()
            cpb.wait()

            def add_row(c, _):
                def add_vec(v, _):
                    sl = pl.ds(v * 16, 16)
                    bufa[c, sl] = bufa[c, sl] + bufb[c, sl]
                    return 0
                lax.fori_loop(0, lanes, add_vec, 0, unroll=4)
                return 0

            lax.fori_loop(0, ch, add_row, 0)
            pltpu.sync_copy(bufa, y_hbm.at[pl.ds(base + i * ch, ch)])
            return 0

        lax.fori_loop(0, nchunk, body, 0)

    return k(outs, pos_even, pos_odd)


def kernel(x, topK_indices, topK_scores, W1, b1, W2, b2):
    perm, src_pad, pos_pad, owner, pp = _route(topK_indices)
    scores_pad = (jnp.zeros((_PP,), jnp.float32)
                  .at[pp].set(topK_scores.reshape(-1)[perm])[:, None])
    pos2 = pos_pad.reshape(_TOK, _KSEL)
    xs = _sc_gather(x, src_pad)
    H = _run_phase_a(xs, W1, b1, owner)
    outs = _run_phase_b(H, W2, b2, scores_pad, owner)
    return _sc_combine(outs, pos2[:, 0], pos2[:, 1])


# speedup vs baseline: 1.6081x; 1.6081x over previous
import functools

import jax
import jax.numpy as jnp
from jax import lax
from jax.experimental import pallas as pl
from jax.experimental.pallas import tpu as pltpu
from jax.experimental.pallas import tpu_sc as plsc

_E = 8
_KSEL = 2
_TOK = 2048
_D = 2048
_F = 4096
_P = _TOK * _KSEL

_TM = 512
_NT = _P // _TM
_S = _NT + _E - 1
_FB = 1024
_J = _F // _FB


def _route(topK_indices):
    flat = topK_indices.reshape(-1).astype(jnp.int32)
    perm = jnp.argsort(flat, stable=True).astype(jnp.int32)
    srcrow = perm // _KSEL
    pos = jnp.zeros((_P,), jnp.int32).at[perm].set(
        jnp.arange(_P, dtype=jnp.int32))
    counts = jnp.bincount(flat, length=_E).astype(jnp.int32)
    start = jnp.concatenate([jnp.zeros((1,), jnp.int32),
                             jnp.cumsum(counts)[:-1].astype(jnp.int32)])
    end = start + counts
    first_t = start // _TM
    last_t = jnp.maximum(end - 1, start) // _TM
    span = jnp.where(counts > 0, last_t - first_t + 1, 0)
    cum = jnp.cumsum(span)
    sids = jnp.arange(_S, dtype=jnp.int32)
    eid = jnp.searchsorted(cum, sids, side="right").astype(jnp.int32)
    valid = sids < cum[-1]
    eidc = jnp.minimum(eid, _E - 1)
    prev = jnp.where(eidc > 0, cum[jnp.maximum(eidc - 1, 0)], 0).astype(jnp.int32)
    tid = first_t[eidc] + (sids - prev)
    tid = jnp.where(valid, tid, _NT - 1).astype(jnp.int32)
    st = jnp.where(valid, start[eidc], 0).astype(jnp.int32)
    en = jnp.where(valid, end[eidc], 0).astype(jnp.int32)
    eidf = jnp.where(valid, eidc, 0).astype(jnp.int32)
    prev_t = jnp.concatenate([jnp.full((1,), -1, jnp.int32), tid[:-1]])
    fv = (tid != prev_t).astype(jnp.int32)
    return perm, srcrow, pos, tid, eidf, st, en, fv


def _sc_gather(x, srcrow):
    info = plsc.get_sparse_core_info()
    nw = info.num_cores * info.num_subcores
    bpw = _P // nw
    ch = 16
    nchunk = bpw // ch
    mesh = plsc.VectorSubcoreMesh(core_axis_name="c", subcore_axis_name="s")

    @functools.partial(
        pl.kernel, mesh=mesh,
        out_type=jax.ShapeDtypeStruct((_P, _D), jnp.float32),
        scratch_types=[
            pltpu.VMEM((bpw,), jnp.int32),
            pltpu.VMEM((ch, _D), jnp.float32),
            pltpu.VMEM((ch, _D), jnp.float32),
            pltpu.SemaphoreType.DMA,
            pltpu.SemaphoreType.DMA,
        ],
    )
    def k(x_hbm, idx_hbm, out_hbm, idx_v, buf0, buf1, sem0, sem1):
        wid = lax.axis_index("s") * info.num_cores + lax.axis_index("c")
        base = wid * bpw
        pltpu.sync_copy(idx_hbm.at[pl.ds(base, bpw)], idx_v)
        bufs = (buf0, buf1)
        sems = (sem0, sem1)
        pltpu.async_copy(x_hbm.at[idx_v.at[pl.ds(0, ch)]], buf0, sem0)

        def body(i, _):
            sl = i % 2

            @pl.when(i + 1 < nchunk)
            def _():
                nsl = (i + 1) % 2
                for b in range(2):
                    @pl.when(nsl == b)
                    def _():
                        pltpu.async_copy(
                            x_hbm.at[idx_v.at[pl.ds((i + 1) * ch, ch)]],
                            bufs[b], sems[b])

            for b in range(2):
                @pl.when(sl == b)
                def _():
                    pltpu.make_async_copy(
                        x_hbm.at[idx_v.at[pl.ds(0, ch)]], bufs[b], sems[b]).wait()
                    pltpu.sync_copy(bufs[b], out_hbm.at[pl.ds(base + i * ch, ch)])
            return 0

        lax.fori_loop(0, nchunk, body, 0)

    return k(x, srcrow)


def _mlp_body(tid_ref, eid_ref, st_ref, en_ref, fv_ref,
              x_ref, w1_ref, b1_ref, w2_ref, b2_ref, sc_ref, o_ref):
    s = pl.program_id(0)
    j = pl.program_id(1)

    @pl.when((fv_ref[s] == 1) & (j == 0))
    def _():
        o_ref[...] = jnp.zeros_like(o_ref)

    row0 = tid_ref[s] * _TM
    rows = row0 + lax.broadcasted_iota(jnp.int32, (_TM, 1), 0)
    mask = (rows >= st_ref[s]) & (rows < en_ref[s])
    x = x_ref[...].astype(jnp.bfloat16)
    h = jnp.dot(x, w1_ref[0].astype(jnp.bfloat16),
                preferred_element_type=jnp.float32)
    h = jnp.maximum(h + b1_ref[0], 0.0)
    contrib = jnp.dot(h.astype(jnp.bfloat16), w2_ref[0].astype(jnp.bfloat16),
                      preferred_element_type=jnp.float32)
    contrib = contrib + jnp.where(j == 0, b2_ref[0], 0.0)
    contrib = contrib * sc_ref[...]
    o_ref[...] += jnp.where(mask, contrib, 0.0)


def _tc_grouped_mlp(xs, scores_sorted, W1, b1, W2, b2, tid, eid, st, en, fv):
    grid_spec = pltpu.PrefetchScalarGridSpec(
        num_scalar_prefetch=5,
        grid=(_S, _J),
        in_specs=[
            pl.BlockSpec((_TM, _D), lambda s, j, t, e, a, b, f: (t[s], 0)),
            pl.BlockSpec((1, _D, _FB),
                         lambda s, j, t, e, a, b, f:
                         (e[s], 0, jnp.where(s % 2 == 1, _J - 1 - j, j))),
            pl.BlockSpec((1, 1, _FB),
                         lambda s, j, t, e, a, b, f:
                         (e[s], 0, jnp.where(s % 2 == 1, _J - 1 - j, j))),
            pl.BlockSpec((1, _FB, _D),
                         lambda s, j, t, e, a, b, f:
                         (e[s], jnp.where(s % 2 == 1, _J - 1 - j, j), 0)),
            pl.BlockSpec((1, 1, _D), lambda s, j, t, e, a, b, f: (e[s], 0, 0)),
            pl.BlockSpec((_TM, 1), lambda s, j, t, e, a, b, f: (t[s], 0)),
        ],
        out_specs=pl.BlockSpec((_TM, _D), lambda s, j, t, e, a, b, f: (t[s], 0)),
    )
    return pl.pallas_call(
        _mlp_body,
        grid_spec=grid_spec,
        out_shape=jax.ShapeDtypeStruct((_P, _D), jnp.float32),
        compiler_params=pltpu.CompilerParams(
            dimension_semantics=("arbitrary", "arbitrary")),
    )(tid, eid, st, en, fv, xs, W1, b1.reshape(_E, 1, _F),
      W2, b2.reshape(_E, 1, _D), scores_sorted)


def _sc_combine(outs, pos_even, pos_odd):
    info = plsc.get_sparse_core_info()
    nw = info.num_cores * info.num_subcores
    tpw = _TOK // nw
    ch = 16
    nchunk = tpw // ch
    lanes = _D // 16
    mesh = plsc.VectorSubcoreMesh(core_axis_name="c", subcore_axis_name="s")

    @functools.partial(
        pl.kernel, mesh=mesh,
        out_type=jax.ShapeDtypeStruct((_TOK, _D), jnp.float32),
        scratch_types=[
            pltpu.VMEM((tpw,), jnp.int32),
            pltpu.VMEM((tpw,), jnp.int32),
            pltpu.VMEM((ch, _D), jnp.float32),
            pltpu.VMEM((ch, _D), jnp.float32),
            pltpu.SemaphoreType.DMA,
            pltpu.SemaphoreType.DMA,
        ],
    )
    def k(rows_hbm, pe_hbm, po_hbm, y_hbm, pe_v, po_v, bufa, bufb, sema, semb):
        wid = lax.axis_index("s") * info.num_cores + lax.axis_index("c")
        base = wid * tpw
        pltpu.sync_copy(pe_hbm.at[pl.ds(base, tpw)], pe_v)
        pltpu.sync_copy(po_hbm.at[pl.ds(base, tpw)], po_v)

        def body(i, _):
            cpa = pltpu.async_copy(
                rows_hbm.at[pe_v.at[pl.ds(i * ch, ch)]], bufa, sema)
            cpb = pltpu.async_copy(
                rows_hbm.at[po_v.at[pl.ds(i * ch, ch)]], bufb, semb)
            cpa.wait()
            cpb.wait()

            def add_row(c, _):
                def add_vec(v, _):
                    sl = pl.ds(v * 16, 16)
                    bufa[c, sl] = bufa[c, sl] + bufb[c, sl]
                    return 0
                lax.fori_loop(0, lanes, add_vec, 0, unroll=4)
                return 0

            lax.fori_loop(0, ch, add_row, 0)
            pltpu.sync_copy(bufa, y_hbm.at[pl.ds(base + i * ch, ch)])
            return 0

        lax.fori_loop(0, nchunk, body, 0)

    return k(outs, pos_even, pos_odd)


def kernel(x, topK_indices, topK_scores, W1, b1, W2, b2):
    perm, srcrow, pos, tid, eid, st, en, fv = _route(topK_indices)
    scores_sorted = topK_scores.reshape(-1)[perm][:, None]
    pos2 = pos.reshape(_TOK, _KSEL)
    xs = _sc_gather(x, srcrow)
    outs = _tc_grouped_mlp(xs, scores_sorted, W1, b1, W2, b2,
                           tid, eid, st, en, fv)
    return _sc_combine(outs, pos2[:, 0], pos2[:, 1])
